# Initial kernel scaffold; baseline (speedup 1.0000x reference)
import jax
import jax.numpy as jnp
from jax.experimental import pallas as pl


def kernel(feats_node, edge_index, feats_graph, W1, al1, ar1, b1, W2, al2, ar2, b2, W3, al3, ar3, b3, score_w, score_b, lin1_w, lin1_b, lin2_w, lin2_b, lin3_w, lin3_b):
    raise NotImplementedError("write your pallas kernel here")


# speedup vs baseline: 77.2528x; 77.2528x over previous
import functools

import jax
import jax.numpy as jnp
from jax import lax
from jax.experimental import pallas as pl
from jax.experimental.pallas import tpu as pltpu
from jax.experimental.pallas import tpu_sc as plsc

N = 10000
E = 320000
HD = 128
H = 8
D = 16
EXTRA = 16

NWORK = 32
EPW = E // NWORK
WN1 = 1000
NWIN1 = EPW // WN1
WN2 = 200
NWIN2 = EPW // WN2

BN = 1000
GRID = N // BN

_f32 = jnp.float32

_SELU_A = 1.6732632423543772848170429916717
_SELU_S = 1.0507009873554804934193349852946


def _selu(x):
    return _SELU_S * jnp.where(x > 0, x, _SELU_A * (jnp.exp(x) - 1.0))


def _dense_body_first(x_ref, w_ref, alw_ref, arw_ref,
                      h_ref, el_ref, er_ref, em_ref, emax_s):
    _dense_common(x_ref[...], w_ref, alw_ref, arw_ref,
                  h_ref, el_ref, er_ref, em_ref, emax_s)


def _dense_body_mid(p0_ref, p1_ref, b_ref, w_ref, alw_ref, arw_ref,
                    h_ref, el_ref, er_ref, em_ref, emax_s):
    x = _selu(p0_ref[...] + p1_ref[...] + b_ref[...])
    _dense_common(x, w_ref, alw_ref, arw_ref,
                  h_ref, el_ref, er_ref, em_ref, emax_s)


def _dense_common(x, w_ref, alw_ref, arw_ref,
                  h_ref, el_ref, er_ref, em_ref, emax_s):
    i = pl.program_id(0)
    h = jnp.dot(x, w_ref[...], preferred_element_type=_f32, precision=lax.Precision.HIGHEST)
    el = jnp.dot(h, alw_ref[...], preferred_element_type=_f32, precision=lax.Precision.HIGHEST)
    er = jnp.dot(h, arw_ref[...], preferred_element_type=_f32, precision=lax.Precision.HIGHEST)
    h_ref[...] = h
    el_ref[...] = el
    er_ref[...] = er
    bm = jnp.max(el, axis=0, keepdims=True)

    @pl.when(i == 0)
    def _():
        emax_s[...] = bm

    @pl.when(i > 0)
    def _():
        emax_s[...] = jnp.maximum(emax_s[...], bm)

    @pl.when(i == pl.num_programs(0) - 1)
    def _():
        em_ref[...] = emax_s[...]


def _dense_call(x_or_parts, w, alw, arw):
    row = pl.BlockSpec((BN, HD), lambda i: (i, 0))
    full = lambda s: pl.BlockSpec(s, lambda i: (0, 0))
    out_shape = [
        jax.ShapeDtypeStruct((N, HD), _f32),
        jax.ShapeDtypeStruct((N, H), _f32),
        jax.ShapeDtypeStruct((N, H), _f32),
        jax.ShapeDtypeStruct((1, H), _f32),
    ]
    out_specs = [row,
                 pl.BlockSpec((BN, H), lambda i: (i, 0)),
                 pl.BlockSpec((BN, H), lambda i: (i, 0)),
                 full((1, H))]
    scratch = [pltpu.VMEM((1, H), _f32)]
    if len(x_or_parts) == 1:
        body = _dense_body_first
        in_specs = [row, full((HD, HD)), full((HD, H)), full((HD, H))]
        args = (x_or_parts[0], w, alw, arw)
    else:
        body = _dense_body_mid
        in_specs = [row, row, full((1, HD)),
                    full((HD, HD)), full((HD, H)), full((HD, H))]
        args = (*x_or_parts, w, alw, arw)
    return pl.pallas_call(
        body, grid=(GRID,), in_specs=in_specs, out_specs=out_specs,
        out_shape=out_shape, scratch_shapes=scratch)(*args)


_MESH = plsc.VectorSubcoreMesh(core_axis_name="c", subcore_axis_name="s")


@functools.partial(
    pl.kernel, mesh=_MESH,
    compiler_params=pltpu.CompilerParams(use_tc_tiling_on_sc=False),
    out_type=[jax.ShapeDtypeStruct((E, 16), _f32),
              jax.ShapeDtypeStruct((2, N, 16), _f32)],
    scratch_types=[
        pltpu.VMEM((WN1,), jnp.int32),
        pltpu.VMEM((WN1,), jnp.int32),
        pltpu.VMEM((WN1, 16), _f32),
        pltpu.VMEM((WN1, 16), _f32),
        pltpu.VMEM((WN1, 16), _f32),
        pltpu.VMEM_SHARED((N, 16), _f32),
        pltpu.SemaphoreType.DMA,
        pltpu.SemaphoreType.DMA,
    ])
def _sc_pass1(atab, btab, srch, dsth, zer16, ex_o, den_o,
              srcv, dstv, arows, brows, exrows, dacc, sem1, sem2):
    c = lax.axis_index("c")
    s = lax.axis_index("s")
    wid = s * 2 + c

    @pl.when(s == 0)
    def _():
        pltpu.sync_copy(zer16, dacc)

    plsc.subcore_barrier()

    lane = lax.iota(jnp.int32, 16)
    idx8 = (lane & 7) + 8
    msk = lane < 8

    def win(w, carry):
        base = pl.multiple_of(wid * EPW + w * WN1, 8)
        pltpu.sync_copy(srch.at[pl.ds(base, WN1)], srcv)
        pltpu.sync_copy(dsth.at[pl.ds(base, WN1)], dstv)
        cp1 = pltpu.async_copy(atab.at[srcv], arows, sem1)
        cp2 = pltpu.async_copy(btab.at[dstv], brows, sem2)
        cp1.wait()
        cp2.wait()

        def edge(e, cy):
            a = arows[e, :]
            b = brows[e, :]
            t = a + b
            ee = jnp.where(t > 0, t, 0.2 * t)
            cs = b.at[idx8].get(mode="promise_in_bounds")
            exv = jnp.exp(ee - cs)
            exrows[e, :] = jnp.where(msk, exv, 0.0)
            return cy

        lax.fori_loop(0, WN1, edge, 0)
        pltpu.sync_copy(exrows, ex_o.at[pl.ds(base, WN1)])
        pltpu.sync_copy(exrows, dacc.at[dstv], add=True)
        return carry

    lax.fori_loop(0, NWIN1, win, 0)
    plsc.subcore_barrier()

    @pl.when(s == 0)
    def _():
        pltpu.sync_copy(dacc, den_o.at[c])


@functools.partial(
    pl.kernel, mesh=_MESH,
    compiler_params=pltpu.CompilerParams(use_tc_tiling_on_sc=False),
    out_type=jax.ShapeDtypeStruct((2, N, HD), _f32),
    scratch_types=[
        pltpu.VMEM((WN2,), jnp.int32),
        pltpu.VMEM((WN2,), jnp.int32),
        pltpu.VMEM((WN2, 16), _f32),
        pltpu.VMEM((WN2, 16), _f32),
        pltpu.VMEM((WN2, HD), _f32),
        pltpu.VMEM_SHARED((N, HD), _f32),
        pltpu.SemaphoreType.DMA,
        pltpu.SemaphoreType.DMA,
    ])
def _sc_pass2(hh, rdtab, exh, srch, dsth, zer128, out_o,
              srcv, dstv, exrows, rdrows, hrows, oacc, sem1, sem2):
    c = lax.axis_index("c")
    s = lax.axis_index("s")
    wid = s * 2 + c

    @pl.when(s == 0)
    def _():
        pltpu.sync_copy(zer128, oacc)

    plsc.subcore_barrier()

    lane = lax.iota(jnp.int32, 16)

    def win(w, carry):
        base = pl.multiple_of(wid * EPW + w * WN2, 8)
        pltpu.sync_copy(srch.at[pl.ds(base, WN2)], srcv)
        pltpu.sync_copy(dsth.at[pl.ds(base, WN2)], dstv)
        pltpu.sync_copy(exh.at[pl.ds(base, WN2)], exrows)
        cp1 = pltpu.async_copy(rdtab.at[dstv], rdrows, sem1)
        cp2 = pltpu.async_copy(hh.at[srcv], hrows, sem2)
        cp1.wait()
        cp2.wait()

        def edge(e, cy):
            alpha = exrows[e, :] * rdrows[e, :]
            for hh8 in range(H):
                ah = alpha.at[lane * 0 + hh8].get(mode="promise_in_bounds")
                hrows[e, pl.ds(hh8 * 16, 16)] = (
                    hrows[e, pl.ds(hh8 * 16, 16)] * ah)
            return cy

        lax.fori_loop(0, WN2, edge, 0)
        pltpu.sync_copy(hrows, oacc.at[dstv], add=True)
        return carry

    lax.fori_loop(0, NWIN2, win, 0)
    plsc.subcore_barrier()

    @pl.when(s == 0)
    def _():
        pltpu.sync_copy(oacc, out_o.at[c])


def _readout_body(p0_ref, p1_ref, b_ref, swt_ref, sb_ref, fg_ref,
                  l1a_ref, l1b_ref, l1bb_ref, l2w_ref, l2b_ref,
                  l3w_ref, l3b_ref, out_ref, accx, accw):
    i = pl.program_id(0)
    x = _selu(p0_ref[...] + p1_ref[...] + b_ref[...])
    sc = jnp.sum(x * swt_ref[...], axis=1, keepdims=True) + sb_ref[...]
    w = jax.nn.sigmoid(sc)
    bx = jnp.sum(w * x, axis=0, keepdims=True)
    bw = jnp.sum(w)

    @pl.when(i == 0)
    def _():
        accx[...] = bx
        accw[0, 0] = bw

    @pl.when(i > 0)
    def _():
        accx[...] = accx[...] + bx
        accw[0, 0] = accw[0, 0] + bw

    @pl.when(i == pl.num_programs(0) - 1)
    def _():
        emb = accx[...] / (accw[0, 0] + 1e-9)
        y = (jnp.dot(emb, l1a_ref[...], preferred_element_type=_f32, precision=lax.Precision.HIGHEST)
             + jnp.dot(fg_ref[...], l1b_ref[...], preferred_element_type=_f32, precision=lax.Precision.HIGHEST)
             + l1bb_ref[...])
        y = _selu(y)
        y = _selu(jnp.dot(y, l2w_ref[...], preferred_element_type=_f32, precision=lax.Precision.HIGHEST)
                        + l2b_ref[...])
        out_ref[...] = (jnp.dot(y, l3w_ref[...], preferred_element_type=_f32, precision=lax.Precision.HIGHEST)
                        + l3b_ref[...])


def _readout_call(p0, p1, b3, score_w, score_b, feats_graph,
                  lin1_w, lin1_b, lin2_w, lin2_b, lin3_w, lin3_b):
    row = pl.BlockSpec((BN, HD), lambda i: (i, 0))
    full = lambda s: pl.BlockSpec(s, lambda i: (0, 0))
    swt = score_w.reshape(1, HD)
    l1a = lin1_w[:HD]
    l1b = lin1_w[HD:]
    return pl.pallas_call(
        _readout_body, grid=(GRID,),
        in_specs=[row, row, full((1, HD)), full((1, HD)), full((1, 1)),
                  full((1, EXTRA)), full((HD, 2 * D)), full((EXTRA, 2 * D)),
                  full((1, 2 * D)), full((2 * D, D)), full((1, D)),
                  full((D, 1)), full((1, 1))],
        out_specs=full((1, 1)),
        out_shape=jax.ShapeDtypeStruct((1, 1), _f32),
        scratch_shapes=[pltpu.VMEM((1, HD), _f32),
                        pltpu.SMEM((1, 1), _f32)],
    )(p0, p1, b3, swt, score_b.reshape(1, 1), feats_graph,
      l1a, l1b, lin1_b.reshape(1, 2 * D), lin2_w, lin2_b.reshape(1, D),
      lin3_w, lin3_b.reshape(1, 1))


def _attn_mat(a):
    sel = (jnp.arange(HD)[:, None] // D == jnp.arange(H)[None, :])
    return sel.astype(_f32) * a.reshape(HD)[:, None]


def kernel(feats_node, edge_index, feats_graph, W1, al1, ar1, b1,
           W2, al2, ar2, b2, W3, al3, ar3, b3, score_w, score_b,
           lin1_w, lin1_b, lin2_w, lin2_b, lin3_w, lin3_b):
    src = edge_index[0]
    dst = edge_index[1]
    z16 = jnp.zeros((N, 16), _f32)
    z128 = jnp.zeros((N, HD), _f32)

    parts = (feats_node,)
    layers = [(W1, al1, ar1, None), (W2, al2, ar2, b1), (W3, al3, ar3, b2)]
    for W, al, ar, bprev in layers:
        if bprev is None:
            h, el, er, elmax = _dense_call(parts, W, _attn_mat(al),
                                           _attn_mat(ar))
        else:
            h, el, er, elmax = _dense_call(
                (parts[0], parts[1], bprev.reshape(1, HD)),
                W, _attn_mat(al), _attn_mat(ar))
        atab = jnp.concatenate([el, el], axis=1)
        cb = jax.nn.leaky_relu(elmax + er, 0.2)
        btab = jnp.concatenate([er, cb], axis=1)
        ex, denp = _sc_pass1(atab, btab, src, dst, z16)
        rd = 1.0 / (denp[0, :, :H] + denp[1, :, :H] + 1e-9)
        rdtab = jnp.concatenate([rd, rd], axis=1)
        outp = _sc_pass2(h, rdtab, ex, src, dst, z128)
        parts = (outp[0], outp[1])

    return _readout_call(parts[0], parts[1], b3.reshape(1, HD), score_w,
                         score_b, feats_graph, lin1_w, lin1_b, lin2_w,
                         lin2_b, lin3_w, lin3_b)
